# Initial kernel scaffold; baseline (speedup 1.0000x reference)
import jax
import jax.numpy as jnp
from jax.experimental import pallas as pl


def kernel(x, batch):
    raise NotImplementedError("write your pallas kernel here")


# speedup vs baseline: 4.7372x; 4.7372x over previous
import functools

import jax
import jax.numpy as jnp
from jax.experimental import pallas as pl
from jax.experimental.pallas import tpu as pltpu

NSEG = 512
N = 320000
D = 128
BLK = 1280


def _tc_body(batch_ref, x_ref, out_ref):
    i = pl.program_id(0)
    b = batch_ref[0, 0, :]
    onehot = (
        jax.lax.broadcasted_iota(jnp.int32, (NSEG, BLK), 0) == b[None, :]
    ).astype(jnp.float32)
    part = jax.lax.dot_general(
        onehot, x_ref[...], (((1,), (0,)), ((), ())),
        preferred_element_type=jnp.float32,
    )

    @pl.when(i == 0)
    def _():
        out_ref[...] = part

    @pl.when(i > 0)
    def _():
        out_ref[...] += part


def kernel(x, batch):
    nblk = N // BLK
    batch3 = batch.astype(jnp.int32).reshape(nblk, 1, BLK)
    out = pl.pallas_call(
        _tc_body,
        grid=(nblk,),
        in_specs=[
            pl.BlockSpec((1, 1, BLK), lambda i: (i, 0, 0)),
            pl.BlockSpec((BLK, D), lambda i: (i, 0)),
        ],
        out_specs=pl.BlockSpec((NSEG, D), lambda i: (0, 0)),
        out_shape=jax.ShapeDtypeStruct((NSEG, D), jnp.float32),
    )(batch3, x)
    return out


# speedup vs baseline: 4.9418x; 1.0432x over previous
import functools

import jax
import jax.numpy as jnp
from jax import lax
from jax.experimental import pallas as pl
from jax.experimental.pallas import tpu as pltpu
from jax.experimental.pallas import tpu_sc as plsc

NSEG = 512
N = 320000
D = 128
DV = D // 16

NW = 32
ROWS_W = N // NW
C = 400
NCHUNK = ROWS_W // C
G = C // 16


def _sc_body(x_hbm, b_hbm, out_hbm, xbuf, ibuf, plane):
    cid = lax.axis_index("c")
    sid = lax.axis_index("s")
    wid = sid * 2 + cid
    base = wid * ROWS_W

    zero = jnp.zeros((16,), jnp.float32)

    def zrow(r, carry):
        prow = plane.at[r]
        for j in range(DV):
            prow[pl.ds(16 * j, 16)] = zero
        return carry

    lax.fori_loop(0, NSEG, zrow, 0)

    pltpu.sync_copy(b_hbm.at[pl.ds(base, C)], ibuf)
    prev0 = ibuf[pl.ds(0, 16)][0]
    carry0 = (prev0,) + (zero,) * DV

    def chunk(k, carry):
        off = base + k * C
        pltpu.sync_copy(x_hbm.at[pl.ds(off, C)], xbuf)
        pltpu.sync_copy(b_hbm.at[pl.ds(off, C)], ibuf)

        def group(g, cr):
            prev = cr[0]
            acc = cr[1:]
            idvec = ibuf[pl.ds(g * 16, 16)]
            for i in range(16):
                seg = idvec[i]
                changed = seg != prev

                @pl.when(changed)
                def _(prev=prev, acc=acc):
                    prow = plane.at[prev]
                    for j in range(DV):
                        prow[pl.ds(16 * j, 16)] = acc[j]

                keep = jnp.where(changed, 0.0, 1.0).astype(jnp.float32)
                xrow = xbuf.at[g * 16 + i]
                acc = tuple(
                    acc[j] * keep + xrow[pl.ds(16 * j, 16)] for j in range(DV)
                )
                prev = seg
            return (prev,) + acc

        return lax.fori_loop(0, G, group, carry)

    carry = lax.fori_loop(0, NCHUNK, chunk, carry0)
    prev = carry[0]
    acc = carry[1:]
    prow = plane.at[prev]
    for j in range(DV):
        prow[pl.ds(16 * j, 16)] = acc[j]

    pltpu.sync_copy(plane, out_hbm.at[wid])


def _combine_body(p_ref, o_ref):
    o_ref[...] = jnp.sum(p_ref[...], axis=0)


def kernel(x, batch):
    b32 = batch.astype(jnp.int32)
    sc = pl.kernel(
        _sc_body,
        out_type=jax.ShapeDtypeStruct((NW, NSEG, D), jnp.float32),
        mesh=plsc.VectorSubcoreMesh(core_axis_name="c", subcore_axis_name="s"),
        scratch_types=[
            pltpu.VMEM((C, D), jnp.float32),
            pltpu.VMEM((C,), jnp.int32),
            pltpu.VMEM((NSEG, D), jnp.float32),
        ],
    )
    partials = sc(x, b32)
    out = pl.pallas_call(
        _combine_body,
        grid=(4,),
        in_specs=[pl.BlockSpec((NW, NSEG // 4, D), lambda i: (0, i, 0))],
        out_specs=pl.BlockSpec((NSEG // 4, D), lambda i: (i, 0)),
        out_shape=jax.ShapeDtypeStruct((NSEG, D), jnp.float32),
    )(partials)
    return out


# speedup vs baseline: 6.9793x; 1.4123x over previous
import functools

import jax
import jax.numpy as jnp
from jax import lax
from jax.experimental import pallas as pl
from jax.experimental.pallas import tpu as pltpu
from jax.experimental.pallas import tpu_sc as plsc

NSEG = 512
N = 320000
D = 128
DV = D // 16

NW = 32
ROWS_W = N // NW
C = 80
NCHUNK = ROWS_W // C
G = C // 16


def _sc_body(x_hbm, b_hbm, out_hbm, xbuf, ids, plane, sems):
    cid = lax.axis_index("c")
    sid = lax.axis_index("s")
    wid = sid * 2 + cid
    base = wid * ROWS_W

    zero = jnp.zeros((16,), jnp.float32)

    def dma_x(k, slot):
        return pltpu.make_async_copy(
            x_hbm.at[pl.ds(base + k * C, C)], xbuf.at[slot], sems.at[slot]
        )

    idcp = pltpu.make_async_copy(
        b_hbm.at[pl.ds(base, ROWS_W)], ids, sems.at[2]
    )
    idcp.start()
    dma_x(0, 0).start()
    dma_x(1, 1).start()

    def zrow(r, carry):
        prow = plane.at[r]
        for j in range(DV):
            prow[pl.ds(16 * j, 16)] = zero
        return carry

    lax.fori_loop(0, NSEG, zrow, 0)

    idcp.wait()
    prev0 = ids[pl.ds(0, 16)][0]
    carry0 = (prev0,) + (zero,) * DV

    def groups(k, slot, carry):
        xb = xbuf.at[slot]

        def group(g, cr):
            prev = cr[0]
            acc = cr[1:]
            idvec = ids[pl.ds(k * C + g * 16, 16)]
            for i in range(16):
                seg = idvec[i]
                changed = seg != prev

                @pl.when(changed)
                def _(prev=prev, acc=acc):
                    prow = plane.at[prev]
                    for j in range(DV):
                        prow[pl.ds(16 * j, 16)] = acc[j]

                keep = jnp.where(changed, 0.0, 1.0).astype(jnp.float32)
                xrow = xb.at[g * 16 + i]
                acc = tuple(
                    acc[j] * keep + xrow[pl.ds(16 * j, 16)] for j in range(DV)
                )
                prev = seg
            return (prev,) + acc

        return lax.fori_loop(0, G, group, carry)

    def pair(p, carry):
        k0 = 2 * p
        dma_x(k0, 0).wait()
        carry = groups(k0, 0, carry)
        dma_x(k0 + 2, 0).start()

        k1 = k0 + 1
        dma_x(k1, 1).wait()
        carry = groups(k1, 1, carry)

        @pl.when(p < (NCHUNK - 1) // 2 - 1)
        def _():
            dma_x(k1 + 2, 1).start()

        return carry

    carry = lax.fori_loop(0, (NCHUNK - 1) // 2, pair, carry0)
    kt = NCHUNK - 1
    dma_x(kt, 0).wait()
    carry = groups(kt, 0, carry)

    prev = carry[0]
    acc = carry[1:]
    prow = plane.at[prev]
    for j in range(DV):
        prow[pl.ds(16 * j, 16)] = acc[j]

    pltpu.sync_copy(plane, out_hbm.at[wid])


def _combine_body(p_ref, o_ref):
    o_ref[...] = jnp.sum(p_ref[...], axis=0)


def kernel(x, batch):
    b32 = batch.astype(jnp.int32)
    sc = pl.kernel(
        _sc_body,
        out_type=jax.ShapeDtypeStruct((NW, NSEG, D), jnp.float32),
        mesh=plsc.VectorSubcoreMesh(core_axis_name="c", subcore_axis_name="s"),
        scratch_types=[
            pltpu.VMEM((2, C, D), jnp.float32),
            pltpu.VMEM((ROWS_W,), jnp.int32),
            pltpu.VMEM((NSEG, D), jnp.float32),
            pltpu.SemaphoreType.DMA((3,)),
        ],
    )
    partials = sc(x, b32)
    out = pl.pallas_call(
        _combine_body,
        grid=(4,),
        in_specs=[pl.BlockSpec((NW, NSEG // 4, D), lambda i: (0, i, 0))],
        out_specs=pl.BlockSpec((NSEG // 4, D), lambda i: (i, 0)),
        out_shape=jax.ShapeDtypeStruct((NSEG, D), jnp.float32),
    )(partials)
    return out
